# baseline (speedup 1.0000x reference)
import functools

import jax
import jax.numpy as jnp
from jax.experimental import pallas as pl
from jax.experimental.pallas import tpu as pltpu

B, P, T, C = 4, 50000, 64, 80
ALPHA, GAMMA, EPS = 0.25, 2.0, 1e-6
BP = 16384
NJ = -(-P // BP)
T2 = T + 8
BIG = 1e9


def _body(yt_ref, bt4_ref, btT_ref, ypT_ref, bpT_ref, ptT_ref, cfT_ref,
          out_ref, bcol_ref, w_ref, w2_ref):
    f32 = jnp.float32
    bf16 = jnp.bfloat16

    @pl.when(pl.program_id(1) == 0)
    def _per_batch():
        bt4 = bt4_ref[0]
        btT = btT_ref[0]
        a_col = (bt4[:, 2:3] - bt4[:, 0:1]) * (bt4[:, 3:4] - bt4[:, 1:2])
        a_row = (btT[2:3, :] - btT[0:1, :]) * (btT[3:4, :] - btT[1:2, :])
        it_r = jax.lax.broadcasted_iota(jnp.int32, (T, T), 0)
        it_c = jax.lax.broadcasted_iota(jnp.int32, (T, T), 1)
        less = (a_row < a_col) | ((a_row == a_col) & (it_c < it_r))
        rank = jnp.sum(less.astype(f32), axis=1, keepdims=True)
        fake_box = jnp.concatenate(
            [jnp.full((8, 2), -BIG, f32), jnp.full((8, 2), BIG, f32)],
            axis=1)
        boxes = jnp.concatenate([bt4, fake_box], axis=0)
        it8 = jax.lax.broadcasted_iota(jnp.int32, (8, 1), 0)
        rank_ext = jnp.concatenate(
            [rank, jnp.where(it8 < 1, f32(T), f32(127.0))], axis=0)
        bcol_ref[:, 0:4] = boxes
        bcol_ref[:, 4:5] = rank_ext
        hi = btT.astype(bf16)
        r1 = btT - hi.astype(f32)
        mid = r1.astype(bf16)
        lo = (r1 - mid.astype(f32)).astype(bf16)
        w_ref[...] = jnp.concatenate(
            [jnp.concatenate([hi, mid, lo, jnp.zeros((4, T), bf16)], axis=0),
             jnp.zeros((16, T2 - T), bf16)], axis=1)
        ytT = jnp.transpose(yt_ref[0], (1, 0))
        e0 = (jax.lax.broadcasted_iota(jnp.int32, (C, 1), 0) < 1)
        ext = jnp.concatenate(
            [e0.astype(bf16), jnp.zeros((C, T2 - T - 1), bf16)], axis=1)
        w2_ref[...] = jnp.concatenate(
            [jnp.concatenate([ytT.astype(bf16), ext], axis=1),
             jnp.zeros((128 - C, T2), bf16)], axis=0)

    bx1 = bcol_ref[:, 0:1]
    by1 = bcol_ref[:, 1:2]
    bx2 = bcol_ref[:, 2:3]
    by2 = bcol_ref[:, 3:4]
    rank_col = bcol_ref[:, 4:5]

    x = ptT_ref[0:1, :]
    y = ptT_ref[1:2, :]

    inside = (x > bx1) & (y > by1) & (bx2 > x) & (by2 > y)
    key = jnp.where(inside, rank_col, f32(128.0))
    kmin = jnp.min(key, axis=0, keepdims=True)
    posf = (kmin < f32(T)).astype(f32)
    onehot = (key == kmin).astype(bf16)

    selc = jax.lax.dot_general(w_ref[...], onehot,
                               (((1,), (0,)), ((), ())),
                               preferred_element_type=f32)
    btx1 = selc[0:1] + selc[4:5] + selc[8:9]
    bty1 = selc[1:2] + selc[5:6] + selc[9:10]
    btx2 = selc[2:3] + selc[6:7] + selc[10:11]
    bty2 = selc[3:4] + selc[7:8] + selc[11:12]

    lo_ = jnp.maximum(x - btx1, EPS)
    to_ = jnp.maximum(y - bty1, EPS)
    ro_ = jnp.maximum(btx2 - x, EPS)
    bo_ = jnp.maximum(bty2 - y, EPS)
    cent = jnp.sqrt((jnp.minimum(lo_, ro_) / jnp.maximum(lo_, ro_))
                    * (jnp.minimum(to_, bo_) / jnp.maximum(to_, bo_)))
    conf_t = jnp.where(posf > 0.0, cent, 0.0)

    bpT = bpT_ref[0]
    px1 = bpT[0:1]
    py1 = bpT[1:2]
    px2 = bpT[2:3]
    py2 = bpT[3:4]
    wi = jnp.maximum(jnp.minimum(px2, btx2) - jnp.maximum(px1, btx1), 0.0)
    hi_ = jnp.maximum(jnp.minimum(py2, bty2) - jnp.maximum(py1, bty1), 0.0)
    inter = wi * hi_
    area_p = jnp.maximum(px2 - px1, 0.0) * jnp.maximum(py2 - py1, 0.0)
    area_t = jnp.maximum(btx2 - btx1, 0.0) * jnp.maximum(bty2 - bty1, 0.0)
    iou = inter / (area_p + area_t - inter + 1e-7)
    bbox_pp = jnp.sum(jnp.where(posf > 0.0, 1.0 - iou, 0.0),
                      axis=(0, 1), keepdims=True)

    cfr = cfT_ref[0]
    cpr = jnp.clip(cfr, EPS, 1.0 - EPS)
    conf_bce = -(conf_t * jnp.log(cpr)
                 + (1.0 - conf_t) * jnp.log(1.0 - cpr))
    conf_pp = jnp.sum(jnp.where(posf > 0.0, conf_bce, 0.0),
                      axis=(0, 1), keepdims=True)

    n_pp = jnp.sum(posf, axis=(0, 1), keepdims=True)

    yext = jax.lax.dot_general(w2_ref[...], onehot,
                               (((1,), (0,)), ((), ())),
                               preferred_element_type=f32)
    mask = yext[0:C, :] > 0.5
    p = jnp.sqrt(jnp.maximum(ypT_ref[0] * cfr, EPS))
    p = jnp.minimum(p, 1.0 - EPS)
    q = 1.0 - p
    pe = jnp.where(mask, p, q)
    om = jnp.where(mask, q, p)
    focal = jnp.where(mask, -ALPHA, ALPHA - 1.0) * om * om * jnp.log(pe)
    limit = P - pl.program_id(1) * BP
    lmask = jax.lax.broadcasted_iota(jnp.int32, (1, BP), 1) < limit
    class_pp = jnp.sum(jnp.where(lmask, focal, 0.0),
                       axis=(0, 1), keepdims=True)

    out_ref[0, 0:1, :] = class_pp
    out_ref[0, 1:2, :] = bbox_pp
    out_ref[0, 2:3, :] = conf_pp
    out_ref[0, 3:4, :] = n_pp


def _pallas_args():
    return dict(
        grid=(B, NJ),
        in_specs=[
            pl.BlockSpec((1, T, C), lambda b, j: (b, 0, 0)),
            pl.BlockSpec((1, T, 4), lambda b, j: (b, 0, 0)),
            pl.BlockSpec((1, 4, T), lambda b, j: (b, 0, 0)),
            pl.BlockSpec((1, C, BP), lambda b, j: (b, 0, j)),
            pl.BlockSpec((1, 4, BP), lambda b, j: (b, 0, j)),
            pl.BlockSpec((2, BP), lambda b, j: (0, j)),
            pl.BlockSpec((1, 1, BP), lambda b, j: (b, 0, j)),
        ],
        out_specs=pl.BlockSpec((1, 4, 1), lambda b, j: (b * NJ + j, 0, 0)),
        out_shape=jax.ShapeDtypeStruct((B * NJ, 4, 1), jnp.float32),
        scratch_shapes=[
            pltpu.VMEM((T2, 8), jnp.float32),
            pltpu.VMEM((16, T2), jnp.bfloat16),
            pltpu.VMEM((128, T2), jnp.bfloat16),
        ],
    )


@functools.partial(jax.jit, static_argnames=())
def kernel(y_true, bbox_true, y_pred, bbox_pred, points, conf_pred):
    btT = jnp.transpose(bbox_true, (0, 2, 1))
    ypT = jnp.transpose(y_pred, (0, 2, 1))
    bpT = jnp.transpose(bbox_pred, (0, 2, 1))
    ptT = jnp.concatenate(
        [jnp.transpose(points, (1, 0)),
         jnp.full((2, NJ * BP - P), -1.0, jnp.float32)], axis=1)
    cfT = jnp.reshape(conf_pred, (B, 1, P))
    partials = pl.pallas_call(_body, **_pallas_args())(
        y_true, bbox_true, btT, ypT, bpT, ptT, cfT)
    sums = jnp.sum(partials[..., 0], axis=0)
    n_pos = jnp.maximum(sums[3], 1.0)
    return jnp.stack([sums[0], sums[1], sums[2]]) / n_pos


# speedup vs baseline: 1.0813x; 1.0813x over previous
import functools

import jax
import jax.numpy as jnp
from jax.experimental import pallas as pl
from jax.experimental.pallas import tpu as pltpu

B, P, T, C = 4, 50000, 64, 80
ALPHA, GAMMA, EPS = 0.25, 2.0, 1e-6
BP = 8192
NJ = -(-P // BP)
T2 = T + 8
BIG = 1e9


def _body(yt_ref, bt4_ref, btT_ref, ypT_ref, bpT_ref, ptT_ref, cfT_ref,
          out_ref, bcol_ref, w_ref, w2_ref):
    f32 = jnp.float32
    bf16 = jnp.bfloat16

    @pl.when(pl.program_id(1) == 0)
    def _per_batch():
        bt4 = bt4_ref[0]
        btT = btT_ref[0]
        a_col = (bt4[:, 2:3] - bt4[:, 0:1]) * (bt4[:, 3:4] - bt4[:, 1:2])
        a_row = (btT[2:3, :] - btT[0:1, :]) * (btT[3:4, :] - btT[1:2, :])
        it_r = jax.lax.broadcasted_iota(jnp.int32, (T, T), 0)
        it_c = jax.lax.broadcasted_iota(jnp.int32, (T, T), 1)
        less = (a_row < a_col) | ((a_row == a_col) & (it_c < it_r))
        rank = jnp.sum(less.astype(f32), axis=1, keepdims=True)
        fake_box = jnp.concatenate(
            [jnp.full((8, 2), -BIG, f32), jnp.full((8, 2), BIG, f32)],
            axis=1)
        boxes = jnp.concatenate([bt4, fake_box], axis=0)
        it8 = jax.lax.broadcasted_iota(jnp.int32, (8, 1), 0)
        rank_ext = jnp.concatenate(
            [rank, jnp.where(it8 < 1, f32(T), f32(127.0))], axis=0)
        bcol_ref[:, 0:4] = boxes
        bcol_ref[:, 4:5] = rank_ext
        hi = btT.astype(bf16)
        r1 = btT - hi.astype(f32)
        mid = r1.astype(bf16)
        lo = (r1 - mid.astype(f32)).astype(bf16)
        w_ref[...] = jnp.concatenate(
            [jnp.concatenate([hi, mid, lo, jnp.zeros((4, T), bf16)], axis=0),
             jnp.zeros((16, T2 - T), bf16)], axis=1)
        ytT = jnp.transpose(yt_ref[0], (1, 0))
        e0 = (jax.lax.broadcasted_iota(jnp.int32, (C, 1), 0) < 1)
        ext = jnp.concatenate(
            [e0.astype(bf16), jnp.zeros((C, T2 - T - 1), bf16)], axis=1)
        w2_ref[...] = jnp.concatenate(
            [jnp.concatenate([ytT.astype(bf16), ext], axis=1),
             jnp.zeros((128 - C, T2), bf16)], axis=0)

    bx1 = bcol_ref[:, 0:1]
    by1 = bcol_ref[:, 1:2]
    bx2 = bcol_ref[:, 2:3]
    by2 = bcol_ref[:, 3:4]
    rank_col = bcol_ref[:, 4:5]

    x = ptT_ref[0:1, :]
    y = ptT_ref[1:2, :]

    inside = (x > bx1) & (y > by1) & (bx2 > x) & (by2 > y)
    key = jnp.where(inside, rank_col, f32(128.0))
    kmin = jnp.min(key, axis=0, keepdims=True)
    posf = (kmin < f32(T)).astype(f32)
    onehot = (key == kmin).astype(bf16)

    selc = jax.lax.dot_general(w_ref[...], onehot,
                               (((1,), (0,)), ((), ())),
                               preferred_element_type=f32)
    btx1 = selc[0:1] + selc[4:5] + selc[8:9]
    bty1 = selc[1:2] + selc[5:6] + selc[9:10]
    btx2 = selc[2:3] + selc[6:7] + selc[10:11]
    bty2 = selc[3:4] + selc[7:8] + selc[11:12]

    lo_ = jnp.maximum(x - btx1, EPS)
    to_ = jnp.maximum(y - bty1, EPS)
    ro_ = jnp.maximum(btx2 - x, EPS)
    bo_ = jnp.maximum(bty2 - y, EPS)
    cent = jnp.sqrt((jnp.minimum(lo_, ro_) / jnp.maximum(lo_, ro_))
                    * (jnp.minimum(to_, bo_) / jnp.maximum(to_, bo_)))
    conf_t = jnp.where(posf > 0.0, cent, 0.0)

    bpT = bpT_ref[0]
    px1 = bpT[0:1]
    py1 = bpT[1:2]
    px2 = bpT[2:3]
    py2 = bpT[3:4]
    wi = jnp.maximum(jnp.minimum(px2, btx2) - jnp.maximum(px1, btx1), 0.0)
    hi_ = jnp.maximum(jnp.minimum(py2, bty2) - jnp.maximum(py1, bty1), 0.0)
    inter = wi * hi_
    area_p = jnp.maximum(px2 - px1, 0.0) * jnp.maximum(py2 - py1, 0.0)
    area_t = jnp.maximum(btx2 - btx1, 0.0) * jnp.maximum(bty2 - bty1, 0.0)
    iou = inter / (area_p + area_t - inter + 1e-7)
    bbox_pp = jnp.sum(jnp.where(posf > 0.0, 1.0 - iou, 0.0),
                      axis=(0, 1), keepdims=True)

    cfr = cfT_ref[0]
    cpr = jnp.clip(cfr, EPS, 1.0 - EPS)
    conf_bce = -(conf_t * jnp.log(cpr)
                 + (1.0 - conf_t) * jnp.log(1.0 - cpr))
    conf_pp = jnp.sum(jnp.where(posf > 0.0, conf_bce, 0.0),
                      axis=(0, 1), keepdims=True)

    n_pp = jnp.sum(posf, axis=(0, 1), keepdims=True)

    yext = jax.lax.dot_general(w2_ref[...], onehot,
                               (((1,), (0,)), ((), ())),
                               preferred_element_type=f32)
    mask = yext[0:C, :] > 0.5
    p = jnp.sqrt(jnp.maximum(ypT_ref[0] * cfr, EPS))
    p = jnp.minimum(p, 1.0 - EPS)
    q = 1.0 - p
    pe = jnp.where(mask, p, q)
    om = jnp.where(mask, q, p)
    focal = jnp.where(mask, -ALPHA, ALPHA - 1.0) * om * om * jnp.log(pe)
    limit = P - pl.program_id(1) * BP
    lmask = jax.lax.broadcasted_iota(jnp.int32, (1, BP), 1) < limit
    class_pp = jnp.sum(jnp.where(lmask, focal, 0.0),
                       axis=(0, 1), keepdims=True)

    out_ref[0, 0:1, :] = class_pp
    out_ref[0, 1:2, :] = bbox_pp
    out_ref[0, 2:3, :] = conf_pp
    out_ref[0, 3:4, :] = n_pp


def _pallas_args():
    return dict(
        grid=(B, NJ),
        in_specs=[
            pl.BlockSpec((1, T, C), lambda b, j: (b, 0, 0)),
            pl.BlockSpec((1, T, 4), lambda b, j: (b, 0, 0)),
            pl.BlockSpec((1, 4, T), lambda b, j: (b, 0, 0)),
            pl.BlockSpec((1, C, BP), lambda b, j: (b, 0, j)),
            pl.BlockSpec((1, 4, BP), lambda b, j: (b, 0, j)),
            pl.BlockSpec((2, BP), lambda b, j: (0, j)),
            pl.BlockSpec((1, 1, BP), lambda b, j: (b, 0, j)),
        ],
        out_specs=pl.BlockSpec((1, 4, 1), lambda b, j: (b * NJ + j, 0, 0)),
        out_shape=jax.ShapeDtypeStruct((B * NJ, 4, 1), jnp.float32),
        compiler_params=pltpu.CompilerParams(
            dimension_semantics=("parallel", "arbitrary")),
        scratch_shapes=[
            pltpu.VMEM((T2, 8), jnp.float32),
            pltpu.VMEM((16, T2), jnp.bfloat16),
            pltpu.VMEM((128, T2), jnp.bfloat16),
        ],
    )


@functools.partial(jax.jit, static_argnames=())
def kernel(y_true, bbox_true, y_pred, bbox_pred, points, conf_pred):
    btT = jnp.transpose(bbox_true, (0, 2, 1))
    ypT = jnp.transpose(y_pred, (0, 2, 1))
    bpT = jnp.transpose(bbox_pred, (0, 2, 1))
    ptT = jnp.concatenate(
        [jnp.transpose(points, (1, 0)),
         jnp.full((2, NJ * BP - P), -1.0, jnp.float32)], axis=1)
    cfT = jnp.reshape(conf_pred, (B, 1, P))
    partials = pl.pallas_call(_body, **_pallas_args())(
        y_true, bbox_true, btT, ypT, bpT, ptT, cfT)
    sums = jnp.sum(partials[..., 0], axis=0)
    n_pos = jnp.maximum(sums[3], 1.0)
    return jnp.stack([sums[0], sums[1], sums[2]]) / n_pos
